# baseline (speedup 1.0000x reference)
import functools

import jax
import jax.numpy as jnp
from jax import lax
from jax.experimental import pallas as pl
from jax.experimental.pallas import tpu as pltpu

SCALING = 2.5
TOPK = 8


def _router_block(x_ref, wt_ref, b_ref, eb_ref, idx_ref, w_ref, probs_ref):
    x = x_ref[...]
    wt = wt_ref[...]
    z = jnp.dot(x, wt, preferred_element_type=jnp.float32) + b_ref[...]
    s = jax.nn.sigmoid(z) + eb_ref[...]
    logits = jnp.log(jnp.maximum(s, 1e-12)) * SCALING
    m = jnp.max(logits, axis=-1, keepdims=True)
    e = jnp.exp(logits - m)
    denom = jnp.sum(e, axis=-1, keepdims=True)
    probs = e / denom
    probs_ref[...] = probs

    rows, E = probs.shape
    iota_e = lax.broadcasted_iota(jnp.int32, (rows, E), 1)
    iota_k = lax.broadcasted_iota(jnp.int32, (rows, TOPK), 1)
    vals = jnp.zeros((rows, TOPK), jnp.float32)
    idxs = jnp.zeros((rows, TOPK), jnp.int32)
    cur = probs
    for k in range(TOPK):
        mk = jnp.max(cur, axis=-1, keepdims=True)
        cand = jnp.where(cur == mk, iota_e, E)
        amin = jnp.min(cand, axis=-1, keepdims=True)
        vals = jnp.where(iota_k == k, mk, vals)
        idxs = jnp.where(iota_k == k, amin, idxs)
        cur = jnp.where(iota_e == amin, -jnp.inf, cur)
    wsum = jnp.maximum(jnp.sum(vals, axis=-1, keepdims=True), 1e-12)
    idx_ref[...] = idxs
    w_ref[...] = vals / wsum


@functools.partial(jax.jit, static_argnames=("block_n",))
def _router(x, wt, b2, eb2, block_n=512):
    n, c = x.shape
    e = wt.shape[1]
    grid = (n // block_n,)
    idx, w, probs = pl.pallas_call(
        _router_block,
        grid=grid,
        in_specs=[
            pl.BlockSpec((block_n, c), lambda i: (i, 0)),
            pl.BlockSpec((c, e), lambda i: (0, 0)),
            pl.BlockSpec((1, e), lambda i: (0, 0)),
            pl.BlockSpec((1, e), lambda i: (0, 0)),
        ],
        out_specs=[
            pl.BlockSpec((block_n, TOPK), lambda i: (i, 0)),
            pl.BlockSpec((block_n, TOPK), lambda i: (i, 0)),
            pl.BlockSpec((block_n, e), lambda i: (i, 0)),
        ],
        out_shape=[
            jax.ShapeDtypeStruct((n, TOPK), jnp.int32),
            jax.ShapeDtypeStruct((n, TOPK), jnp.float32),
            jax.ShapeDtypeStruct((n, e), jnp.float32),
        ],
        compiler_params=pltpu.CompilerParams(
            dimension_semantics=("arbitrary",),
        ),
    )(x, wt, b2, eb2)
    return idx, w, probs


def kernel(x, W, b, expert_bias):
    wt = W.T
    b2 = b.reshape(1, -1)
    eb2 = expert_bias.reshape(1, -1)
    idx, w, probs = _router(x, wt, b2, eb2)
    return idx.astype(jnp.int64), w, probs


# speedup vs baseline: 1.1494x; 1.1494x over previous
import functools

import jax
import jax.numpy as jnp
from jax import lax
from jax.experimental import pallas as pl
from jax.experimental.pallas import tpu as pltpu

SCALING = 2.5
TOPK = 8


def _router_block(x_ref, wt_ref, b_ref, eb_ref, idx_ref, w_ref, probs_ref):
    x = x_ref[...]
    wt = wt_ref[...]
    z = jnp.dot(x, wt, preferred_element_type=jnp.float32) + b_ref[...]
    s = jax.nn.sigmoid(z) + eb_ref[...]
    logits = jnp.log(jnp.maximum(s, 1e-12)) * SCALING
    m = jnp.max(logits, axis=-1, keepdims=True)
    e = jnp.exp(logits - m)
    denom = jnp.sum(e, axis=-1, keepdims=True)
    probs = e / denom
    probs_ref[...] = probs

    rows, E = probs.shape
    iota_e = lax.broadcasted_iota(jnp.int32, (rows, E), 1)
    iota_k = lax.broadcasted_iota(jnp.int32, (rows, TOPK), 1)
    vals = jnp.zeros((rows, TOPK), jnp.float32)
    idxs = jnp.zeros((rows, TOPK), jnp.int32)
    cur = probs
    for k in range(TOPK):
        mk = jnp.max(cur, axis=-1, keepdims=True)
        cand = jnp.where(cur == mk, iota_e, E)
        amin = jnp.min(cand, axis=-1, keepdims=True)
        vals = jnp.where(iota_k == k, mk, vals)
        idxs = jnp.where(iota_k == k, amin, idxs)
        cur = jnp.where(iota_e == amin, -jnp.inf, cur)
    wsum = jnp.maximum(jnp.sum(vals, axis=-1, keepdims=True), 1e-12)
    idx_ref[...] = idxs
    w_ref[...] = vals / wsum


@functools.partial(jax.jit, static_argnames=("block_n",))
def _router(x, wt, b2, eb2, block_n=512):
    n, c = x.shape
    e = wt.shape[1]
    grid = (n // block_n,)
    idx, w, probs = pl.pallas_call(
        _router_block,
        grid=grid,
        in_specs=[
            pl.BlockSpec((block_n, c), lambda i: (i, 0)),
            pl.BlockSpec((c, e), lambda i: (0, 0)),
            pl.BlockSpec((1, e), lambda i: (0, 0)),
            pl.BlockSpec((1, e), lambda i: (0, 0)),
        ],
        out_specs=[
            pl.BlockSpec((block_n, TOPK), lambda i: (i, 0)),
            pl.BlockSpec((block_n, TOPK), lambda i: (i, 0)),
            pl.BlockSpec((block_n, e), lambda i: (i, 0)),
        ],
        out_shape=[
            jax.ShapeDtypeStruct((n, TOPK), jnp.int32),
            jax.ShapeDtypeStruct((n, TOPK), jnp.float32),
            jax.ShapeDtypeStruct((n, e), jnp.float32),
        ],
        compiler_params=pltpu.CompilerParams(
            dimension_semantics=("arbitrary",),
        ),
    )(x, wt, b2, eb2)
    return idx, w, probs


def kernel(x, W, b, expert_bias):
    wt = W.T
    b2 = b.reshape(1, -1)
    eb2 = expert_bias.reshape(1, -1)
    idx, w, probs = _router(x, wt, b2, eb2, block_n=1024)
    return idx.astype(jnp.int64), w, probs
